# baseline (speedup 1.0000x reference)
import jax
import jax.numpy as jnp
from jax import lax
from jax.experimental import pallas as pl
from jax.experimental.pallas import tpu as pltpu
from jax.experimental.pallas import tpu_sc as plsc

D = 32
B = 16384
NC = 2
NS = 16
L = 16
NW = NC * NS
BPW = B // NW
CHUNK = 128
NCHUNK = BPW // CHUNK
ROWS_PER_GROW = 128 // D


def _sc_body(w_hbm, idx0_hbm, idx1_hbm, out_hbm,
             idx_v, gidx_v, off_v, rows_v, out_v, sems):
    wid = lax.axis_index("s") * NC + lax.axis_index("c")
    crow = wid * NCHUNK
    pltpu.sync_copy(idx0_hbm.at[pl.ds(crow, NCHUNK)], idx_v.at[0])
    pltpu.sync_copy(idx1_hbm.at[pl.ds(crow, NCHUNK)], idx_v.at[1])

    for f in range(2):
        for j in range(NCHUNK):
            for s in range(CHUNK // L):
                iv = idx_v[f, j, pl.ds(s * L, L)]
                gidx_v[f, j, pl.ds(s * L, L)] = lax.shift_right_logical(iv, 2)
                off_v[f, j, pl.ds(s * L, L)] = lax.shift_left(
                    lax.bitwise_and(iv, 3), 5)

    def fire(j, buf):
        cps = []
        for f in range(2):
            cps.append(pltpu.async_copy(
                w_hbm.at[gidx_v.at[f, j]], rows_v.at[buf, f], sems.at[buf]))
        return cps

    pending = fire(0, 0)
    lanes = lax.iota(jnp.int32, L)

    for j in range(NCHUNK):
        buf = j % 2
        if j + 1 < NCHUNK:
            nxt = fire(j + 1, 1 - buf)
        for c in pending:
            c.wait()

        def block_body(b, carry):
            lrow = b * L + lanes
            off0 = off_v[0, j, pl.ds(b * L, L)]
            off1 = off_v[1, j, pl.ds(b * L, L)]
            acc = jnp.zeros((L,), jnp.float32)
            for d in range(D):
                a0 = plsc.load_gather(rows_v.at[buf, 0], [lrow, off0 + d])
                a1 = plsc.load_gather(rows_v.at[buf, 1], [lrow, off1 + d])
                acc = acc + a0 * a1
            out_v[pl.ds(j * CHUNK + b * L, L)] = acc
            return carry

        lax.fori_loop(0, CHUNK // L, block_body, 0)
        if j + 1 < NCHUNK:
            pending = nxt

    pltpu.sync_copy(out_v, out_hbm.at[pl.ds(wid * BPW, BPW)])


@jax.jit
def kernel(sparse_features, W):
    idx = sparse_features.astype(jnp.int32)
    idx0 = idx[:, 0].reshape(B // CHUNK, CHUNK)
    idx1 = idx[:, 1].reshape(B // CHUNK, CHUNK)
    w4 = W.reshape(W.shape[0] // ROWS_PER_GROW, D * ROWS_PER_GROW)
    mesh = plsc.VectorSubcoreMesh(core_axis_name="c", subcore_axis_name="s")
    out = pl.kernel(
        _sc_body,
        out_type=jax.ShapeDtypeStruct((B,), jnp.float32),
        mesh=mesh,
        compiler_params=pltpu.CompilerParams(needs_layout_passes=False),
        scratch_types=[
            pltpu.VMEM((2, NCHUNK, CHUNK), jnp.int32),
            pltpu.VMEM((2, NCHUNK, CHUNK), jnp.int32),
            pltpu.VMEM((2, NCHUNK, CHUNK), jnp.int32),
            pltpu.VMEM((2, 2, CHUNK, D * ROWS_PER_GROW), jnp.float32),
            pltpu.VMEM((BPW,), jnp.float32),
            pltpu.SemaphoreType.DMA((2,)),
        ],
    )(w4, idx0, idx1)
    return out.reshape(B, 1)


# speedup vs baseline: 1.0808x; 1.0808x over previous
import jax
import jax.numpy as jnp
from jax import lax
from jax.experimental import pallas as pl
from jax.experimental.pallas import tpu as pltpu
from jax.experimental.pallas import tpu_sc as plsc

D = 32
B = 16384
NC = 2
NS = 16
L = 16
NW = NC * NS
BPW = B // NW
CHUNK = 128
NCHUNK = BPW // CHUNK
ROWS_PER_GROW = 128 // D


def _sc_body(w_hbm, idx0_hbm, idx1_hbm, out_hbm,
             idx_v, gidx_v, off_v, rows_v, out_v, sems):
    wid = lax.axis_index("s") * NC + lax.axis_index("c")
    crow = wid * NCHUNK
    pltpu.sync_copy(idx0_hbm.at[pl.ds(crow, NCHUNK)], idx_v.at[0])
    pltpu.sync_copy(idx1_hbm.at[pl.ds(crow, NCHUNK)], idx_v.at[1])

    for f in range(2):
        for j in range(NCHUNK):
            for s in range(CHUNK // L):
                iv = idx_v[f, j, pl.ds(s * L, L)]
                gidx_v[f, j, pl.ds(s * L, L)] = lax.shift_right_logical(iv, 2)
                off_v[f, j, pl.ds(s * L, L)] = lax.shift_left(
                    lax.bitwise_and(iv, 3), 5)

    def fire(j, buf):
        cps = []
        for f in range(2):
            cps.append(pltpu.async_copy(
                w_hbm.at[gidx_v.at[f, j]], rows_v.at[buf, f], sems.at[buf]))
        return cps

    pending = fire(0, 0)
    lanes = lax.iota(jnp.int32, L)

    for j in range(NCHUNK):
        buf = j % 2
        if j + 1 < NCHUNK:
            nxt = fire(j + 1, 1 - buf)
        for c in pending:
            c.wait()

        def block_body(b, carry):
            lrow = b * L + lanes
            off0 = off_v[0, j, pl.ds(b * L, L)]
            off1 = off_v[1, j, pl.ds(b * L, L)]
            acc = jnp.zeros((L,), jnp.float32)
            for d in range(D):
                a0 = plsc.load_gather(rows_v.at[buf, 0], [lrow, off0 + d])
                a1 = plsc.load_gather(rows_v.at[buf, 1], [lrow, off1 + d])
                acc = acc + a0 * a1
            out_v[pl.ds(j * CHUNK + b * L, L)] = acc
            return carry

        lax.fori_loop(0, CHUNK // L, block_body, 0)
        if j + 1 < NCHUNK:
            pending = nxt

    pltpu.sync_copy(out_v, out_hbm.at[pl.ds(wid * BPW, BPW)])


@jax.jit
def kernel(sparse_features, W):
    idx = sparse_features.astype(jnp.int32)
    idx0 = idx[:, 0].reshape(B // CHUNK, CHUNK)
    idx1 = idx[:, 1].reshape(B // CHUNK, CHUNK)
    w4 = (W.T.reshape(D, 250000, ROWS_PER_GROW)
          .transpose(1, 2, 0)
          .reshape(250000, D * ROWS_PER_GROW))
    mesh = plsc.VectorSubcoreMesh(core_axis_name="c", subcore_axis_name="s")
    out = pl.kernel(
        _sc_body,
        out_type=jax.ShapeDtypeStruct((B,), jnp.float32),
        mesh=mesh,
        compiler_params=pltpu.CompilerParams(needs_layout_passes=False),
        scratch_types=[
            pltpu.VMEM((2, NCHUNK, CHUNK), jnp.int32),
            pltpu.VMEM((2, NCHUNK, CHUNK), jnp.int32),
            pltpu.VMEM((2, NCHUNK, CHUNK), jnp.int32),
            pltpu.VMEM((2, 2, CHUNK, D * ROWS_PER_GROW), jnp.float32),
            pltpu.VMEM((BPW,), jnp.float32),
            pltpu.SemaphoreType.DMA((2,)),
        ],
    )(w4, idx0, idx1)
    return out.reshape(B, 1)
